# Initial kernel scaffold; baseline (speedup 1.0000x reference)
import jax
import jax.numpy as jnp
from jax.experimental import pallas as pl


def kernel(micro_price, pre_w, params):
    raise NotImplementedError("write your pallas kernel here")


# speedup vs baseline: 4.1197x; 4.1197x over previous
import functools

import jax
import jax.numpy as jnp
from jax import lax
from jax.experimental import pallas as pl

_ALPHA = 0.2
_NEG_REAL = -9e15
_NEG_PAD = -1e30

_INTERPRET = False


def _elu(v):
    return jnp.where(v > 0, v, jnp.exp(jnp.minimum(v, 0.0)) - 1.0)


def _masked_softmax_rows(e, adjpos, colmask):
    m = jnp.where(adjpos, e, jnp.float32(_NEG_REAL))
    m = jnp.where(colmask, m, jnp.float32(_NEG_PAD))
    mx = jnp.max(m, axis=1, keepdims=True)
    p = jnp.exp(m - mx)
    return p / jnp.sum(p, axis=1, keepdims=True)


def _gat_body(x_ref, prew_ref, wh_ref, a1_ref, a2_ref, wo_ref, ao1_ref,
              ao2_ref, wm1_ref, wm1p_ref, bm1_ref, wm2_ref, bm2_ref, out_ref):
    x = x_ref[0]
    n_pad = x.shape[0]
    mean = jnp.mean(x, axis=1, keepdims=True)
    xc = x - mean
    cov = lax.dot_general(xc, xc, (((1,), (1,)), ((), ())),
                          preferred_element_type=jnp.float32) * (1.0 / 2047.0)
    adjpos = cov > 0.0
    colmask = lax.broadcasted_iota(jnp.int32, (n_pad, n_pad), 1) < 501

    wh2 = jnp.zeros((n_pad, 64), dtype=jnp.float32)
    for h in range(4):
        w_h = wh_ref[h]
        wh = jnp.dot(x, w_h, preferred_element_type=jnp.float32)
        f1 = jnp.dot(wh, a1_ref[h], preferred_element_type=jnp.float32)
        f2t = lax.dot_general(a2_ref[h], wh, (((0,), (1,)), ((), ())),
                              preferred_element_type=jnp.float32)
        e = f1 + f2t
        e = jnp.where(e > 0, e, _ALPHA * e)
        att = _masked_softmax_rows(e, adjpos, colmask)
        hh = _elu(jnp.dot(att, wh, preferred_element_type=jnp.float32))
        wh2 = wh2 + jnp.dot(hh, wo_ref[h], preferred_element_type=jnp.float32)

    f1 = jnp.dot(wh2, ao1_ref[...], preferred_element_type=jnp.float32)
    f2t = lax.dot_general(ao2_ref[...], wh2, (((0,), (1,)), ((), ())),
                          preferred_element_type=jnp.float32)
    e = f1 + f2t
    e = jnp.where(e > 0, e, _ALPHA * e)
    att = _masked_softmax_rows(e, adjpos, colmask)
    hidden = _elu(jnp.dot(att, wh2, preferred_element_type=jnp.float32))

    pre = prew_ref[0]
    h1 = jnp.maximum(
        jnp.dot(hidden, wm1_ref[...], preferred_element_type=jnp.float32)
        + pre * wm1p_ref[...] + bm1_ref[...], 0.0)
    out = jnp.dot(h1, wm2_ref[...], preferred_element_type=jnp.float32) + bm2_ref[0, 0]
    rowmask = lax.broadcasted_iota(jnp.int32, (n_pad, 1), 0) < 501
    m = jnp.where(rowmask, out, jnp.float32(_NEG_PAD))
    mx = jnp.max(m, axis=0, keepdims=True)
    p = jnp.exp(m - mx)
    out_ref[0] = p / jnp.sum(p, axis=0, keepdims=True)


def _score_body(x_ref, w1_ref, b1_ref, w2_ref, b2_ref, out_ref):
    x = x_ref[0, 0]
    h = jnp.maximum(
        jnp.dot(x, w1_ref[...], preferred_element_type=jnp.float32) + b1_ref[...], 0.0)
    z = jnp.dot(h, w2_ref[...], preferred_element_type=jnp.float32) + b2_ref[0, 0]
    out_ref[0, 0] = 1.0 / (1.0 + jnp.exp(-z))


def _topk_body(s_ref, wsel_ref, pre_ref, tp_ref):
    s = s_ref[0]
    n_s, n_t = s.shape
    iota_t = lax.broadcasted_iota(jnp.int32, (n_s, n_t), 1)
    col_k = lax.broadcasted_iota(jnp.int32, (n_s, 16), 1)
    bos = wsel_ref[0] > pre_ref[0]

    smax = s
    smin = s
    tp = jnp.zeros((n_s, 16), dtype=jnp.int32)
    for k in range(16):
        mx = jnp.max(smax, axis=1, keepdims=True)
        sell_idx = jnp.min(jnp.where(smax == mx, iota_t, n_t), axis=1, keepdims=True)
        smax = jnp.where(iota_t == sell_idx, jnp.float32(-jnp.inf), smax)
        mn = jnp.min(smin, axis=1, keepdims=True)
        buy_idx = jnp.min(jnp.where(smin == mn, iota_t, n_t), axis=1, keepdims=True)
        smin = jnp.where(iota_t == buy_idx, jnp.float32(jnp.inf), smin)
        choice = jnp.where(bos, buy_idx, sell_idx)
        tp = jnp.where(col_k == k, choice, tp)
    tp_ref[0] = tp


def kernel(micro_price, pre_w, params):
    b, s, t, f = micro_price.shape
    n = s + 1
    n_pad = 512
    in_feat = t * f

    feat = micro_price.reshape(b, s, in_feat)
    featp = jnp.concatenate(
        [jnp.ones((b, 1, in_feat), jnp.float32), feat,
         jnp.zeros((b, n_pad - n, in_feat), jnp.float32)], axis=1)
    prew_pad = jnp.pad(pre_w, ((0, 0), (0, n_pad - n)))[..., None]

    wheads = jnp.stack([p["W"] for p in params["gat_heads"]])
    a1 = jnp.stack([p["a"][:64] for p in params["gat_heads"]])
    a2 = jnp.stack([p["a"][64:] for p in params["gat_heads"]])
    wout4 = params["gat_out"]["W"].reshape(4, 64, 64)
    ao1 = params["gat_out"]["a"][:64]
    ao2 = params["gat_out"]["a"][64:]
    wm = params["w_mlp"]
    wm1 = wm[0]["W"][:64]
    wm1p = wm[0]["W"][64:65]
    bm1 = wm[0]["b"][None, :]
    wm2 = wm[1]["W"]
    bm2 = wm[1]["b"].reshape(1, 1)
    sc = params["score_mlp"]
    ws1 = sc[0]["W"]
    bs1 = sc[0]["b"][None, :]
    ws2 = sc[1]["W"]
    bs2 = sc[1]["b"].reshape(1, 1)

    def _full(shape):
        return pl.BlockSpec(shape, lambda *_: (0,) * len(shape))

    w3 = pl.pallas_call(
        _gat_body,
        grid=(b,),
        in_specs=[
            pl.BlockSpec((1, n_pad, in_feat), lambda i: (i, 0, 0)),
            pl.BlockSpec((1, n_pad, 1), lambda i: (i, 0, 0)),
            _full((4, 2048, 64)), _full((4, 64, 1)), _full((4, 64, 1)),
            _full((4, 64, 64)), _full((64, 1)), _full((64, 1)),
            _full((64, 64)), _full((1, 64)), _full((1, 64)),
            _full((64, 1)), _full((1, 1)),
        ],
        out_specs=pl.BlockSpec((1, n_pad, 1), lambda i: (i, 0, 0)),
        out_shape=jax.ShapeDtypeStruct((b, n_pad, 1), jnp.float32),
        interpret=_INTERPRET,
    )(featp, prew_pad, wheads, a1, a2, wout4, ao1, ao2,
      wm1, wm1p, bm1, wm2, bm2)

    w = w3[:, :n, 0]

    n_chunk = 4
    rows = s * t // n_chunk
    mp_flat = micro_price.reshape(b, n_chunk, rows, f)
    score_flat = pl.pallas_call(
        _score_body,
        grid=(b, n_chunk),
        in_specs=[
            pl.BlockSpec((1, 1, rows, f), lambda i, j: (i, j, 0, 0)),
            _full((16, 64)), _full((1, 64)), _full((64, 1)), _full((1, 1)),
        ],
        out_specs=pl.BlockSpec((1, 1, rows, 1), lambda i, j: (i, j, 0, 0)),
        out_shape=jax.ShapeDtypeStruct((b, n_chunk, rows, 1), jnp.float32),
        interpret=_INTERPRET,
    )(mp_flat, ws1, bs1, ws2, bs2)
    score = score_flat.reshape(b, s, t)

    w_sel = w3[:, 1:n]
    pre_col = pre_w[:, 1:, None]
    trading_points = pl.pallas_call(
        _topk_body,
        grid=(b,),
        in_specs=[
            pl.BlockSpec((1, s, t), lambda i: (i, 0, 0)),
            pl.BlockSpec((1, s, 1), lambda i: (i, 0, 0)),
            pl.BlockSpec((1, s, 1), lambda i: (i, 0, 0)),
        ],
        out_specs=pl.BlockSpec((1, s, 16), lambda i: (i, 0, 0)),
        out_shape=jax.ShapeDtypeStruct((b, s, 16), jnp.int32),
        interpret=_INTERPRET,
    )(score, w_sel, pre_col)

    return w, trading_points, score


# speedup vs baseline: 4.3387x; 1.0531x over previous
import functools

import jax
import jax.numpy as jnp
from jax import lax
from jax.experimental import pallas as pl

_ALPHA = 0.2
_NEG_REAL = -9e15
_NEG_PAD = -1e30

_INTERPRET = False


def _elu(v):
    return jnp.where(v > 0, v, jnp.exp(jnp.minimum(v, 0.0)) - 1.0)


def _masked_softmax_rows(e, adjpos, colmask):
    m = jnp.where(adjpos, e, jnp.float32(_NEG_REAL))
    m = jnp.where(colmask, m, jnp.float32(_NEG_PAD))
    mx = jnp.max(m, axis=1, keepdims=True)
    p = jnp.exp(m - mx)
    return p / jnp.sum(p, axis=1, keepdims=True)


def _gat_body(x_ref, prew_ref, wh_ref, a1_ref, a2_ref, wo_ref, ao1_ref,
              ao2_ref, wm1_ref, wm1p_ref, bm1_ref, wm2_ref, bm2_ref, out_ref):
    xr = x_ref[0]
    n_pad = 512
    d = xr.shape[1]
    x = jnp.concatenate(
        [jnp.ones((1, d), jnp.float32), xr,
         jnp.zeros((n_pad - 1 - xr.shape[0], d), jnp.float32)], axis=0)
    mean = jnp.mean(x, axis=1, keepdims=True)
    xc = x - mean
    cov = lax.dot_general(xc, xc, (((1,), (1,)), ((), ())),
                          preferred_element_type=jnp.float32) * (1.0 / 2047.0)
    adjpos = cov > 0.0
    colmask = lax.broadcasted_iota(jnp.int32, (n_pad, n_pad), 1) < 501

    wh2 = jnp.zeros((n_pad, 64), dtype=jnp.float32)
    for h in range(4):
        w_h = wh_ref[h]
        wh = jnp.dot(x, w_h, preferred_element_type=jnp.float32)
        f1 = jnp.dot(wh, a1_ref[h], preferred_element_type=jnp.float32)
        f2t = lax.dot_general(a2_ref[h], wh, (((0,), (1,)), ((), ())),
                              preferred_element_type=jnp.float32)
        e = f1 + f2t
        e = jnp.where(e > 0, e, _ALPHA * e)
        att = _masked_softmax_rows(e, adjpos, colmask)
        hh = _elu(jnp.dot(att, wh, preferred_element_type=jnp.float32))
        wh2 = wh2 + jnp.dot(hh, wo_ref[h], preferred_element_type=jnp.float32)

    f1 = jnp.dot(wh2, ao1_ref[...], preferred_element_type=jnp.float32)
    f2t = lax.dot_general(ao2_ref[...], wh2, (((0,), (1,)), ((), ())),
                          preferred_element_type=jnp.float32)
    e = f1 + f2t
    e = jnp.where(e > 0, e, _ALPHA * e)
    att = _masked_softmax_rows(e, adjpos, colmask)
    hidden = _elu(jnp.dot(att, wh2, preferred_element_type=jnp.float32))

    pre = prew_ref[0]
    h1 = jnp.maximum(
        jnp.dot(hidden, wm1_ref[...], preferred_element_type=jnp.float32)
        + pre * wm1p_ref[...] + bm1_ref[...], 0.0)
    out = jnp.dot(h1, wm2_ref[...], preferred_element_type=jnp.float32) + bm2_ref[0, 0]
    rowmask = lax.broadcasted_iota(jnp.int32, (n_pad, 1), 0) < 501
    m = jnp.where(rowmask, out, jnp.float32(_NEG_PAD))
    mx = jnp.max(m, axis=0, keepdims=True)
    p = jnp.exp(m - mx)
    out_ref[0] = p / jnp.sum(p, axis=0, keepdims=True)


def _score_body(x_ref, w1_ref, b1_ref, w2_ref, b2_ref, out_ref):
    x = x_ref[0, 0]
    h = jnp.maximum(
        jnp.dot(x, w1_ref[...], preferred_element_type=jnp.float32) + b1_ref[...], 0.0)
    z = jnp.dot(h, w2_ref[...], preferred_element_type=jnp.float32) + b2_ref[0, 0]
    out_ref[0, 0] = 1.0 / (1.0 + jnp.exp(-z))


def _topk_body(s_ref, wsel_ref, pre_ref, tp_ref):
    s = s_ref[0]
    n_s, n_t = s.shape
    iota_t = lax.broadcasted_iota(jnp.int32, (n_s, n_t), 1)
    col_k = lax.broadcasted_iota(jnp.int32, (n_s, 16), 1)
    bos = wsel_ref[0] > pre_ref[0]

    smax = s
    smin = s
    tp = jnp.zeros((n_s, 16), dtype=jnp.int32)
    for k in range(16):
        mx = jnp.max(smax, axis=1, keepdims=True)
        sell_idx = jnp.min(jnp.where(smax == mx, iota_t, n_t), axis=1, keepdims=True)
        smax = jnp.where(iota_t == sell_idx, jnp.float32(-jnp.inf), smax)
        mn = jnp.min(smin, axis=1, keepdims=True)
        buy_idx = jnp.min(jnp.where(smin == mn, iota_t, n_t), axis=1, keepdims=True)
        smin = jnp.where(iota_t == buy_idx, jnp.float32(jnp.inf), smin)
        choice = jnp.where(bos, buy_idx, sell_idx)
        tp = jnp.where(col_k == k, choice, tp)
    tp_ref[0] = tp


def kernel(micro_price, pre_w, params):
    b, s, t, f = micro_price.shape
    n = s + 1
    n_pad = 512
    in_feat = t * f

    feat = micro_price.reshape(b, s, in_feat)
    prew_pad = jnp.pad(pre_w, ((0, 0), (0, n_pad - n)))[..., None]

    wheads = jnp.stack([p["W"] for p in params["gat_heads"]])
    a1 = jnp.stack([p["a"][:64] for p in params["gat_heads"]])
    a2 = jnp.stack([p["a"][64:] for p in params["gat_heads"]])
    wout4 = params["gat_out"]["W"].reshape(4, 64, 64)
    ao1 = params["gat_out"]["a"][:64]
    ao2 = params["gat_out"]["a"][64:]
    wm = params["w_mlp"]
    wm1 = wm[0]["W"][:64]
    wm1p = wm[0]["W"][64:65]
    bm1 = wm[0]["b"][None, :]
    wm2 = wm[1]["W"]
    bm2 = wm[1]["b"].reshape(1, 1)
    sc = params["score_mlp"]
    ws1 = sc[0]["W"]
    bs1 = sc[0]["b"][None, :]
    ws2 = sc[1]["W"]
    bs2 = sc[1]["b"].reshape(1, 1)

    def _full(shape):
        return pl.BlockSpec(shape, lambda *_: (0,) * len(shape))

    w3 = pl.pallas_call(
        _gat_body,
        grid=(b,),
        in_specs=[
            pl.BlockSpec((1, s, in_feat), lambda i: (i, 0, 0)),
            pl.BlockSpec((1, n_pad, 1), lambda i: (i, 0, 0)),
            _full((4, 2048, 64)), _full((4, 64, 1)), _full((4, 64, 1)),
            _full((4, 64, 64)), _full((64, 1)), _full((64, 1)),
            _full((64, 64)), _full((1, 64)), _full((1, 64)),
            _full((64, 1)), _full((1, 1)),
        ],
        out_specs=pl.BlockSpec((1, n_pad, 1), lambda i: (i, 0, 0)),
        out_shape=jax.ShapeDtypeStruct((b, n_pad, 1), jnp.float32),
        interpret=_INTERPRET,
    )(feat, prew_pad, wheads, a1, a2, wout4, ao1, ao2,
      wm1, wm1p, bm1, wm2, bm2)

    w = w3[:, :n, 0]

    n_chunk = 4
    rows = s * t // n_chunk
    mp_flat = micro_price.reshape(b, n_chunk, rows, f)
    score_flat = pl.pallas_call(
        _score_body,
        grid=(b, n_chunk),
        in_specs=[
            pl.BlockSpec((1, 1, rows, f), lambda i, j: (i, j, 0, 0)),
            _full((16, 64)), _full((1, 64)), _full((64, 1)), _full((1, 1)),
        ],
        out_specs=pl.BlockSpec((1, 1, rows, 1), lambda i, j: (i, j, 0, 0)),
        out_shape=jax.ShapeDtypeStruct((b, n_chunk, rows, 1), jnp.float32),
        interpret=_INTERPRET,
    )(mp_flat, ws1, bs1, ws2, bs2)
    score = score_flat.reshape(b, s, t)

    w_sel = w3[:, 1:n]
    pre_col = pre_w[:, 1:, None]
    trading_points = pl.pallas_call(
        _topk_body,
        grid=(b,),
        in_specs=[
            pl.BlockSpec((1, s, t), lambda i: (i, 0, 0)),
            pl.BlockSpec((1, s, 1), lambda i: (i, 0, 0)),
            pl.BlockSpec((1, s, 1), lambda i: (i, 0, 0)),
        ],
        out_specs=pl.BlockSpec((1, s, 16), lambda i: (i, 0, 0)),
        out_shape=jax.ShapeDtypeStruct((b, s, 16), jnp.int32),
        interpret=_INTERPRET,
    )(score, w_sel, pre_col)

    return w, trading_points, score


# speedup vs baseline: 7.4818x; 1.7245x over previous
import jax
import jax.numpy as jnp
from jax import lax
from jax.experimental import pallas as pl

_ALPHA = 0.2
_NEG_REAL = -9e15
_NEG_PAD = -1e30

_INTERPRET = False


def _elu(v):
    return jnp.where(v > 0, v, jnp.exp(jnp.minimum(v, 0.0)) - 1.0)


def _masked_softmax_rows(e, adjpos, colmask):
    m = jnp.where(adjpos, e, jnp.float32(_NEG_REAL))
    m = jnp.where(colmask, m, jnp.float32(_NEG_PAD))
    mx = jnp.max(m, axis=1, keepdims=True)
    p = jnp.exp(m - mx)
    return p / jnp.sum(p, axis=1, keepdims=True)


def _attention(wh, a1, a2, adjpos, colmask):
    f1 = jnp.dot(wh, a1, preferred_element_type=jnp.float32)
    f2t = lax.dot_general(a2, wh, (((0,), (1,)), ((), ())),
                          preferred_element_type=jnp.float32)
    e = f1 + f2t
    e = jnp.where(e > 0, e, _ALPHA * e)
    att = _masked_softmax_rows(e, adjpos, colmask)
    return jnp.dot(att, wh, preferred_element_type=jnp.float32)


def _body(x_ref, prew_ref, wstack_ref, a1_ref, a2_ref, wo_ref, ao1_ref,
          ao2_ref, wm1_ref, wm1p_ref, bm1_ref, wm2_ref, bm2_ref,
          w1t_ref, b1c_ref, w2t_ref, bs2_ref,
          w_ref, score_ref, tp_ref):
    xp = x_ref[0]
    n_pad, n_s, n_t, n_f = 512, 500, 128, 16

    def plane(f):
        p = xp[f:f + 1, :].reshape(n_s, n_t)
        return jnp.concatenate(
            [jnp.ones((1, n_t), jnp.float32), p,
             jnp.zeros((n_pad - 1 - n_s, n_t), jnp.float32)], axis=0)

    planes = [plane(f) for f in range(n_f)]

    rowsum = planes[0].sum(axis=1, keepdims=True)
    for f in range(1, n_f):
        rowsum = rowsum + planes[f].sum(axis=1, keepdims=True)
    mean = rowsum * (1.0 / (n_t * n_f))

    cov = None
    whh = [None] * 4
    for f in range(n_f):
        xf = planes[f]
        xcf = xf - mean
        c = lax.dot_general(xcf, xcf, (((1,), (1,)), ((), ())),
                            preferred_element_type=jnp.float32)
        cov = c if cov is None else cov + c
        for h in range(4):
            ph = jnp.dot(xf, wstack_ref[h, f], preferred_element_type=jnp.float32)
            whh[h] = ph if whh[h] is None else whh[h] + ph
    cov = cov * (1.0 / (n_t * n_f - 1))

    adjpos = cov > 0.0
    colmask = lax.broadcasted_iota(jnp.int32, (n_pad, n_pad), 1) < (n_s + 1)

    wh2 = None
    for h in range(4):
        hh = _elu(_attention(whh[h], a1_ref[h], a2_ref[h], adjpos, colmask))
        contrib = jnp.dot(hh, wo_ref[h], preferred_element_type=jnp.float32)
        wh2 = contrib if wh2 is None else wh2 + contrib

    hidden = _elu(_attention(wh2, ao1_ref[...], ao2_ref[...], adjpos, colmask))

    pre = prew_ref[0]
    h1 = jnp.maximum(
        jnp.dot(hidden, wm1_ref[...], preferred_element_type=jnp.float32)
        + pre * wm1p_ref[...] + bm1_ref[...], 0.0)
    out = jnp.dot(h1, wm2_ref[...], preferred_element_type=jnp.float32) + bm2_ref[0, 0]
    rowmask = lax.broadcasted_iota(jnp.int32, (n_pad, 1), 0) < (n_s + 1)
    m = jnp.where(rowmask, out, jnp.float32(_NEG_PAD))
    mx = jnp.max(m, axis=0, keepdims=True)
    p = jnp.exp(m - mx)
    w_col = p / jnp.sum(p, axis=0, keepdims=True)
    w_ref[0] = w_col

    chunks = []
    n_chunk = 4
    cols = xp.shape[1] // n_chunk
    for c in range(n_chunk):
        xc = xp[:, c * cols:(c + 1) * cols]
        hs = jnp.maximum(
            jnp.dot(w1t_ref[...], xc, preferred_element_type=jnp.float32)
            + b1c_ref[...], 0.0)
        z = jnp.dot(w2t_ref[...], hs, preferred_element_type=jnp.float32) + bs2_ref[0, 0]
        chunks.append(1.0 / (1.0 + jnp.exp(-z)))
    score = jnp.concatenate(chunks, axis=1).reshape(n_s, n_t)
    score_ref[0] = score

    iota_t = lax.broadcasted_iota(jnp.int32, (n_s, n_t), 1)
    col_k = lax.broadcasted_iota(jnp.int32, (n_s, 16), 1)
    bos = w_col[1:n_s + 1, :] > pre[1:n_s + 1, :]

    smax = score
    smin = score
    tp = jnp.zeros((n_s, 16), dtype=jnp.int32)
    for k in range(16):
        mx = jnp.max(smax, axis=1, keepdims=True)
        sell_idx = jnp.min(jnp.where(smax == mx, iota_t, n_t), axis=1, keepdims=True)
        smax = jnp.where(iota_t == sell_idx, jnp.float32(-jnp.inf), smax)
        mn = jnp.min(smin, axis=1, keepdims=True)
        buy_idx = jnp.min(jnp.where(smin == mn, iota_t, n_t), axis=1, keepdims=True)
        smin = jnp.where(iota_t == buy_idx, jnp.float32(jnp.inf), smin)
        choice = jnp.where(bos, buy_idx, sell_idx)
        tp = jnp.where(col_k == k, choice, tp)
    tp_ref[0] = tp


def kernel(micro_price, pre_w, params):
    b, s, t, f = micro_price.shape
    n = s + 1
    n_pad = 512

    mpt = micro_price.transpose(0, 3, 1, 2).reshape(b, f, s * t)
    prew_pad = jnp.pad(pre_w, ((0, 0), (0, n_pad - n)))[..., None]

    wstack = jnp.stack([p["W"].reshape(t, f, 64).transpose(1, 0, 2)
                        for p in params["gat_heads"]])
    a1 = jnp.stack([p["a"][:64] for p in params["gat_heads"]])
    a2 = jnp.stack([p["a"][64:] for p in params["gat_heads"]])
    wout4 = params["gat_out"]["W"].reshape(4, 64, 64)
    ao1 = params["gat_out"]["a"][:64]
    ao2 = params["gat_out"]["a"][64:]
    wm = params["w_mlp"]
    wm1 = wm[0]["W"][:64]
    wm1p = wm[0]["W"][64:65]
    bm1 = wm[0]["b"][None, :]
    wm2 = wm[1]["W"]
    bm2 = wm[1]["b"].reshape(1, 1)
    sc = params["score_mlp"]
    w1t = sc[0]["W"].T
    b1c = sc[0]["b"][:, None]
    w2t = sc[1]["W"].T
    bs2 = sc[1]["b"].reshape(1, 1)

    def _full(shape):
        return pl.BlockSpec(shape, lambda *_: (0,) * len(shape))

    w3, score, trading_points = pl.pallas_call(
        _body,
        grid=(b,),
        in_specs=[
            pl.BlockSpec((1, f, s * t), lambda i: (i, 0, 0)),
            pl.BlockSpec((1, n_pad, 1), lambda i: (i, 0, 0)),
            _full((4, f, t, 64)), _full((4, 64, 1)), _full((4, 64, 1)),
            _full((4, 64, 64)), _full((64, 1)), _full((64, 1)),
            _full((64, 64)), _full((1, 64)), _full((1, 64)),
            _full((64, 1)), _full((1, 1)),
            _full((64, f)), _full((64, 1)), _full((1, 64)), _full((1, 1)),
        ],
        out_specs=[
            pl.BlockSpec((1, n_pad, 1), lambda i: (i, 0, 0)),
            pl.BlockSpec((1, s, t), lambda i: (i, 0, 0)),
            pl.BlockSpec((1, s, 16), lambda i: (i, 0, 0)),
        ],
        out_shape=[
            jax.ShapeDtypeStruct((b, n_pad, 1), jnp.float32),
            jax.ShapeDtypeStruct((b, s, t), jnp.float32),
            jax.ShapeDtypeStruct((b, s, 16), jnp.int32),
        ],
        interpret=_INTERPRET,
    )(mpt, prew_pad, wstack, a1, a2, wout4, ao1, ao2,
      wm1, wm1p, bm1, wm2, bm2, w1t, b1c, w2t, bs2)

    w = w3[:, :n, 0]
    return w, trading_points, score


# speedup vs baseline: 8.8350x; 1.1809x over previous
import jax
import jax.numpy as jnp
from jax import lax
from jax.experimental import pallas as pl

_ALPHA = 0.2
_NEG_REAL = -9e15
_NEG_PAD = -1e30

_INTERPRET = False


def _elu(v):
    return jnp.where(v > 0, v, jnp.exp(jnp.minimum(v, 0.0)) - 1.0)


def _masked_softmax_rows(e, adjpos, colmask):
    m = jnp.where(adjpos, e, jnp.float32(_NEG_REAL))
    m = jnp.where(colmask, m, jnp.float32(_NEG_PAD))
    mx = jnp.max(m, axis=1, keepdims=True)
    p = jnp.exp(m - mx)
    return p / jnp.sum(p, axis=1, keepdims=True)


def _attention(wh, a1, a2, adjpos, colmask):
    f1 = jnp.dot(wh, a1, preferred_element_type=jnp.float32)
    f2t = lax.dot_general(a2, wh, (((0,), (1,)), ((), ())),
                          preferred_element_type=jnp.float32)
    e = f1 + f2t
    e = jnp.where(e > 0, e, _ALPHA * e)
    att = _masked_softmax_rows(e, adjpos, colmask)
    return jnp.dot(att, wh, preferred_element_type=jnp.float32)


def _body(x_ref, prew_ref, wstack_ref, a1_ref, a2_ref, wo_ref, ao1_ref,
          ao2_ref, wm1_ref, wm1p_ref, bm1_ref, wm2_ref, bm2_ref,
          w1t_ref, b1c_ref, w2t_ref, bs2_ref,
          w_ref, score_ref, tp_ref):
    xp = x_ref[0]
    n_pad, n_s, n_t, n_f = 512, 500, 128, 16

    def plane(f):
        p = xp[f:f + 1, :].reshape(n_s, n_t)
        return jnp.concatenate(
            [jnp.ones((1, n_t), jnp.float32), p,
             jnp.zeros((n_pad - 1 - n_s, n_t), jnp.float32)], axis=0)

    planes = [plane(f) for f in range(n_f)]

    rowsum = planes[0].sum(axis=1, keepdims=True)
    for f in range(1, n_f):
        rowsum = rowsum + planes[f].sum(axis=1, keepdims=True)
    mean = rowsum * (1.0 / (n_t * n_f))

    cov = None
    whh = [None] * 4
    for f in range(n_f):
        xf = planes[f]
        xcf = xf - mean
        c = lax.dot_general(xcf, xcf, (((1,), (1,)), ((), ())),
                            preferred_element_type=jnp.float32)
        cov = c if cov is None else cov + c
        for h in range(4):
            ph = jnp.dot(xf, wstack_ref[h, f], preferred_element_type=jnp.float32)
            whh[h] = ph if whh[h] is None else whh[h] + ph
    cov = cov * (1.0 / (n_t * n_f - 1))

    adjpos = cov > 0.0
    colmask = lax.broadcasted_iota(jnp.int32, (n_pad, n_pad), 1) < (n_s + 1)

    wh2 = None
    for h in range(4):
        hh = _elu(_attention(whh[h], a1_ref[h], a2_ref[h], adjpos, colmask))
        contrib = jnp.dot(hh, wo_ref[h], preferred_element_type=jnp.float32)
        wh2 = contrib if wh2 is None else wh2 + contrib

    hidden = _elu(_attention(wh2, ao1_ref[...], ao2_ref[...], adjpos, colmask))

    pre = prew_ref[0]
    h1 = jnp.maximum(
        jnp.dot(hidden, wm1_ref[...], preferred_element_type=jnp.float32)
        + pre * wm1p_ref[...] + bm1_ref[...], 0.0)
    out = jnp.dot(h1, wm2_ref[...], preferred_element_type=jnp.float32) + bm2_ref[0, 0]
    rowmask = lax.broadcasted_iota(jnp.int32, (n_pad, 1), 0) < (n_s + 1)
    m = jnp.where(rowmask, out, jnp.float32(_NEG_PAD))
    mx = jnp.max(m, axis=0, keepdims=True)
    p = jnp.exp(m - mx)
    w_col = p / jnp.sum(p, axis=0, keepdims=True)
    w_ref[0] = w_col

    chunks = []
    n_chunk = 4
    cols = xp.shape[1] // n_chunk
    for c in range(n_chunk):
        xc = xp[:, c * cols:(c + 1) * cols]
        hs = jnp.maximum(
            jnp.dot(w1t_ref[...], xc, preferred_element_type=jnp.float32)
            + b1c_ref[...], 0.0)
        z = jnp.dot(w2t_ref[...], hs, preferred_element_type=jnp.float32) + bs2_ref[0, 0]
        chunks.append(1.0 / (1.0 + jnp.exp(-z)))
    score = jnp.concatenate(chunks, axis=1).reshape(n_s, n_t)
    score_ref[0] = score

    iota_t = lax.broadcasted_iota(jnp.int32, (n_s, n_t), 1).astype(jnp.float32)
    col_k = lax.broadcasted_iota(jnp.int32, (n_s, 16), 1).astype(jnp.float32)
    bos = w_col[1:n_s + 1, :] > pre[1:n_s + 1, :]

    smax = score
    smin = score
    tp = jnp.zeros((n_s, 16), dtype=jnp.float32)
    for k in range(16):
        mx = jnp.max(smax, axis=1, keepdims=True)
        sell_idx = jnp.min(jnp.where(smax == mx, iota_t, jnp.float32(n_t)),
                           axis=1, keepdims=True)
        smax = jnp.where(iota_t == sell_idx, jnp.float32(-jnp.inf), smax)
        mn = jnp.min(smin, axis=1, keepdims=True)
        buy_idx = jnp.min(jnp.where(smin == mn, iota_t, jnp.float32(n_t)),
                          axis=1, keepdims=True)
        smin = jnp.where(iota_t == buy_idx, jnp.float32(jnp.inf), smin)
        choice = jnp.where(bos, buy_idx, sell_idx)
        tp = jnp.where(col_k == jnp.float32(k), choice, tp)
    tp_ref[0] = tp.astype(jnp.int32)


def kernel(micro_price, pre_w, params):
    b, s, t, f = micro_price.shape
    n = s + 1
    n_pad = 512

    mpt = micro_price.transpose(0, 3, 1, 2).reshape(b, f, s * t)
    prew_pad = jnp.pad(pre_w, ((0, 0), (0, n_pad - n)))[..., None]

    wstack = jnp.stack([p["W"].reshape(t, f, 64).transpose(1, 0, 2)
                        for p in params["gat_heads"]])
    a1 = jnp.stack([p["a"][:64] for p in params["gat_heads"]])
    a2 = jnp.stack([p["a"][64:] for p in params["gat_heads"]])
    wout4 = params["gat_out"]["W"].reshape(4, 64, 64)
    ao1 = params["gat_out"]["a"][:64]
    ao2 = params["gat_out"]["a"][64:]
    wm = params["w_mlp"]
    wm1 = wm[0]["W"][:64]
    wm1p = wm[0]["W"][64:65]
    bm1 = wm[0]["b"][None, :]
    wm2 = wm[1]["W"]
    bm2 = wm[1]["b"].reshape(1, 1)
    sc = params["score_mlp"]
    w1t = sc[0]["W"].T
    b1c = sc[0]["b"][:, None]
    w2t = sc[1]["W"].T
    bs2 = sc[1]["b"].reshape(1, 1)

    def _full(shape):
        return pl.BlockSpec(shape, lambda *_: (0,) * len(shape))

    w3, score, trading_points = pl.pallas_call(
        _body,
        grid=(b,),
        in_specs=[
            pl.BlockSpec((1, f, s * t), lambda i: (i, 0, 0)),
            pl.BlockSpec((1, n_pad, 1), lambda i: (i, 0, 0)),
            _full((4, f, t, 64)), _full((4, 64, 1)), _full((4, 64, 1)),
            _full((4, 64, 64)), _full((64, 1)), _full((64, 1)),
            _full((64, 64)), _full((1, 64)), _full((1, 64)),
            _full((64, 1)), _full((1, 1)),
            _full((64, f)), _full((64, 1)), _full((1, 64)), _full((1, 1)),
        ],
        out_specs=[
            pl.BlockSpec((1, n_pad, 1), lambda i: (i, 0, 0)),
            pl.BlockSpec((1, s, t), lambda i: (i, 0, 0)),
            pl.BlockSpec((1, s, 16), lambda i: (i, 0, 0)),
        ],
        out_shape=[
            jax.ShapeDtypeStruct((b, n_pad, 1), jnp.float32),
            jax.ShapeDtypeStruct((b, s, t), jnp.float32),
            jax.ShapeDtypeStruct((b, s, 16), jnp.int32),
        ],
        interpret=_INTERPRET,
    )(mpt, prew_pad, wstack, a1, a2, wout4, ao1, ao2,
      wm1, wm1p, bm1, wm2, bm2, w1t, b1c, w2t, bs2)

    w = w3[:, :n, 0]
    return w, trading_points, score


# speedup vs baseline: 8.8954x; 1.0068x over previous
import jax
import jax.numpy as jnp
from jax import lax
from jax.experimental import pallas as pl

_ALPHA = 0.2
_NEG_REAL = -9e15
_NEG_PAD = -1e30

_INTERPRET = False


def _elu(v):
    return jnp.where(v > 0, v, jnp.exp(jnp.minimum(v, 0.0)) - 1.0)


def _masked_softmax_rows(e, adjpos, colmask):
    m = jnp.where(adjpos, e, jnp.float32(_NEG_REAL))
    m = jnp.where(colmask, m, jnp.float32(_NEG_PAD))
    mx = jnp.max(m, axis=1, keepdims=True)
    p = jnp.exp(m - mx)
    return p / jnp.sum(p, axis=1, keepdims=True)


def _attention(wh, a1, a2, adjpos, colmask):
    f1 = jnp.dot(wh, a1, preferred_element_type=jnp.float32)
    f2t = lax.dot_general(a2, wh, (((0,), (1,)), ((), ())),
                          preferred_element_type=jnp.float32)
    e = f1 + f2t
    e = jnp.where(e > 0, e, _ALPHA * e)
    att = _masked_softmax_rows(e, adjpos, colmask)
    return jnp.dot(att, wh, preferred_element_type=jnp.float32)


def _body(x_ref, prew_ref, wstack_ref, a1_ref, a2_ref, wo_ref, ao1_ref,
          ao2_ref, wm1_ref, wm1p_ref, bm1_ref, wm2_ref, bm2_ref,
          w1t_ref, b1c_ref, w2t_ref, bs2_ref,
          w_ref, score_ref, tp_ref):
    xp = x_ref[0]
    n_pad, n_s, n_t, n_f = 512, 500, 128, 16

    def plane(f):
        p = xp[f:f + 1, :].reshape(n_s, n_t)
        return jnp.concatenate(
            [jnp.ones((1, n_t), jnp.float32), p,
             jnp.zeros((n_pad - 1 - n_s, n_t), jnp.float32)], axis=0)

    planes = [plane(f) for f in range(n_f)]

    rowsum = planes[0].sum(axis=1, keepdims=True)
    for f in range(1, n_f):
        rowsum = rowsum + planes[f].sum(axis=1, keepdims=True)
    mean = rowsum * (1.0 / (n_t * n_f))

    cov = None
    whh = [None] * 4
    for f in range(n_f):
        xf = planes[f]
        xcf = xf - mean
        c = lax.dot_general(xcf, xcf, (((1,), (1,)), ((), ())),
                            preferred_element_type=jnp.float32)
        cov = c if cov is None else cov + c
        for h in range(4):
            ph = jnp.dot(xf, wstack_ref[h, f], preferred_element_type=jnp.float32)
            whh[h] = ph if whh[h] is None else whh[h] + ph
    cov = cov * (1.0 / (n_t * n_f - 1))

    adjpos = cov > 0.0
    colmask = lax.broadcasted_iota(jnp.int32, (n_pad, n_pad), 1) < (n_s + 1)

    wh2 = None
    for h in range(4):
        hh = _elu(_attention(whh[h], a1_ref[h], a2_ref[h], adjpos, colmask))
        contrib = jnp.dot(hh, wo_ref[h], preferred_element_type=jnp.float32)
        wh2 = contrib if wh2 is None else wh2 + contrib

    hidden = _elu(_attention(wh2, ao1_ref[...], ao2_ref[...], adjpos, colmask))

    pre = prew_ref[0]
    h1 = jnp.maximum(
        jnp.dot(hidden, wm1_ref[...], preferred_element_type=jnp.float32)
        + pre * wm1p_ref[...] + bm1_ref[...], 0.0)
    out = jnp.dot(h1, wm2_ref[...], preferred_element_type=jnp.float32) + bm2_ref[0, 0]
    rowmask = lax.broadcasted_iota(jnp.int32, (n_pad, 1), 0) < (n_s + 1)
    m = jnp.where(rowmask, out, jnp.float32(_NEG_PAD))
    mx = jnp.max(m, axis=0, keepdims=True)
    p = jnp.exp(m - mx)
    w_col = p / jnp.sum(p, axis=0, keepdims=True)
    w_ref[0] = w_col

    chunks = []
    n_chunk = 4
    cols = xp.shape[1] // n_chunk
    for c in range(n_chunk):
        xc = xp[:, c * cols:(c + 1) * cols]
        hs = jnp.maximum(
            jnp.dot(w1t_ref[...], xc, preferred_element_type=jnp.float32)
            + b1c_ref[...], 0.0)
        z = jnp.dot(w2t_ref[...], hs, preferred_element_type=jnp.float32) + bs2_ref[0, 0]
        chunks.append(1.0 / (1.0 + jnp.exp(-z)))
    score = jnp.concatenate(chunks, axis=1).reshape(n_s, n_t)
    score_ref[0] = score

    iota_t = lax.broadcasted_iota(jnp.int32, (n_s, n_t), 1).astype(jnp.float32)
    col_k = lax.broadcasted_iota(jnp.int32, (n_s, 16), 1).astype(jnp.float32)
    bos = w_col[1:n_s + 1, :] > pre[1:n_s + 1, :]

    smax = score
    smin = score
    tp = jnp.zeros((n_s, 16), dtype=jnp.float32)
    for k in range(16):
        mx = jnp.max(smax, axis=1, keepdims=True)
        sell_idx = jnp.min(jnp.where(smax == mx, iota_t, jnp.float32(n_t)),
                           axis=1, keepdims=True)
        smax = jnp.where(iota_t == sell_idx, jnp.float32(-jnp.inf), smax)
        mn = jnp.min(smin, axis=1, keepdims=True)
        buy_idx = jnp.min(jnp.where(smin == mn, iota_t, jnp.float32(n_t)),
                          axis=1, keepdims=True)
        smin = jnp.where(iota_t == buy_idx, jnp.float32(jnp.inf), smin)
        choice = jnp.where(bos, buy_idx, sell_idx)
        tp = jnp.where(col_k == jnp.float32(k), choice, tp)
    tp_ref[0] = tp.astype(jnp.int32)


def kernel(micro_price, pre_w, params):
    b, s, t, f = micro_price.shape
    n = s + 1
    n_pad = 512

    mpt = micro_price.transpose(0, 3, 1, 2).reshape(b, f, s * t)
    prew_pad = jnp.pad(pre_w, ((0, 0), (0, n_pad - n)))[..., None]

    wall = jnp.stack([p["W"] for p in params["gat_heads"]])
    wstack = wall.reshape(4, t, f, 64).transpose(0, 2, 1, 3)
    aall = jnp.stack([p["a"] for p in params["gat_heads"]])
    a1 = aall[:, :64]
    a2 = aall[:, 64:]
    wout4 = params["gat_out"]["W"].reshape(4, 64, 64)
    ao1 = params["gat_out"]["a"][:64]
    ao2 = params["gat_out"]["a"][64:]
    wm = params["w_mlp"]
    wm1 = wm[0]["W"][:64]
    wm1p = wm[0]["W"][64:65]
    bm1 = wm[0]["b"][None, :]
    wm2 = wm[1]["W"]
    bm2 = wm[1]["b"].reshape(1, 1)
    sc = params["score_mlp"]
    w1t = sc[0]["W"].T
    b1c = sc[0]["b"][:, None]
    w2t = sc[1]["W"].T
    bs2 = sc[1]["b"].reshape(1, 1)

    def _full(shape):
        return pl.BlockSpec(shape, lambda *_: (0,) * len(shape))

    w3, score, trading_points = pl.pallas_call(
        _body,
        grid=(b,),
        in_specs=[
            pl.BlockSpec((1, f, s * t), lambda i: (i, 0, 0)),
            pl.BlockSpec((1, n_pad, 1), lambda i: (i, 0, 0)),
            _full((4, f, t, 64)), _full((4, 64, 1)), _full((4, 64, 1)),
            _full((4, 64, 64)), _full((64, 1)), _full((64, 1)),
            _full((64, 64)), _full((1, 64)), _full((1, 64)),
            _full((64, 1)), _full((1, 1)),
            _full((64, f)), _full((64, 1)), _full((1, 64)), _full((1, 1)),
        ],
        out_specs=[
            pl.BlockSpec((1, n_pad, 1), lambda i: (i, 0, 0)),
            pl.BlockSpec((1, s, t), lambda i: (i, 0, 0)),
            pl.BlockSpec((1, s, 16), lambda i: (i, 0, 0)),
        ],
        out_shape=[
            jax.ShapeDtypeStruct((b, n_pad, 1), jnp.float32),
            jax.ShapeDtypeStruct((b, s, t), jnp.float32),
            jax.ShapeDtypeStruct((b, s, 16), jnp.int32),
        ],
        interpret=_INTERPRET,
    )(mpt, prew_pad, wstack, a1, a2, wout4, ao1, ao2,
      wm1, wm1p, bm1, wm2, bm2, w1t, b1c, w2t, bs2)

    w = w3[:, :n, 0]
    return w, trading_points, score


# speedup vs baseline: 9.1594x; 1.0297x over previous
import jax
import jax.numpy as jnp
from jax import lax
from jax.experimental import pallas as pl

_ALPHA = 0.2
_NEG_REAL = -9e15
_NEG_PAD = -1e30

_INTERPRET = False


def _elu(v):
    return jnp.where(v > 0, v, jnp.exp(jnp.minimum(v, 0.0)) - 1.0)


def _masked_softmax_rows(e, adjpos, negfill):
    m = jnp.where(adjpos, e, negfill)
    mx = jnp.max(m, axis=1, keepdims=True)
    p = jnp.exp(m - mx)
    return p / jnp.sum(p, axis=1, keepdims=True)


def _attention(wh, a1, a2, adjpos, negfill):
    f1 = jnp.dot(wh, a1, preferred_element_type=jnp.float32)
    f2t = lax.dot_general(a2, wh, (((0,), (1,)), ((), ())),
                          preferred_element_type=jnp.float32)
    e = f1 + f2t
    e = jnp.where(e > 0, e, _ALPHA * e)
    att = _masked_softmax_rows(e, adjpos, negfill)
    return jnp.dot(att, wh, preferred_element_type=jnp.float32)


def _body(x_ref, prew_ref, wstack_ref, a1_ref, a2_ref, wo_ref, ao1_ref,
          ao2_ref, wm1_ref, wm1p_ref, bm1_ref, wm2_ref, bm2_ref,
          w1t_ref, b1c_ref, w2t_ref, bs2_ref,
          w_ref, score_ref, tp_ref):
    xp = x_ref[0]
    n_pad, n_s, n_t, n_f = 512, 500, 128, 16

    def plane(f):
        p = xp[f:f + 1, :].reshape(n_s, n_t)
        return jnp.concatenate(
            [jnp.ones((1, n_t), jnp.float32), p,
             jnp.zeros((n_pad - 1 - n_s, n_t), jnp.float32)], axis=0)

    x2 = jnp.concatenate([plane(f) for f in range(n_f)], axis=1)
    mean = jnp.sum(x2, axis=1, keepdims=True) * (1.0 / (n_t * n_f))
    xc = x2 - mean
    cov = lax.dot_general(xc, xc, (((1,), (1,)), ((), ())),
                          preferred_element_type=jnp.float32) * (1.0 / (n_t * n_f - 1))

    adjpos = cov > 0.0
    colmask = lax.broadcasted_iota(jnp.int32, (n_pad, n_pad), 1) < (n_s + 1)
    negfill = jnp.where(colmask, jnp.float32(_NEG_REAL), jnp.float32(_NEG_PAD))

    wh2 = None
    for h in range(4):
        whh = jnp.dot(x2, wstack_ref[h], preferred_element_type=jnp.float32)
        hh = _elu(_attention(whh, a1_ref[h], a2_ref[h], adjpos, negfill))
        contrib = jnp.dot(hh, wo_ref[h], preferred_element_type=jnp.float32)
        wh2 = contrib if wh2 is None else wh2 + contrib

    hidden = _elu(_attention(wh2, ao1_ref[...], ao2_ref[...], adjpos, negfill))

    pre = prew_ref[0]
    h1 = jnp.maximum(
        jnp.dot(hidden, wm1_ref[...], preferred_element_type=jnp.float32)
        + pre * wm1p_ref[...] + bm1_ref[...], 0.0)
    out = jnp.dot(h1, wm2_ref[...], preferred_element_type=jnp.float32) + bm2_ref[0, 0]
    rowmask = lax.broadcasted_iota(jnp.int32, (n_pad, 1), 0) < (n_s + 1)
    m = jnp.where(rowmask, out, jnp.float32(_NEG_PAD))
    mx = jnp.max(m, axis=0, keepdims=True)
    p = jnp.exp(m - mx)
    w_col = p / jnp.sum(p, axis=0, keepdims=True)
    w_ref[0] = w_col

    chunks = []
    n_chunk = 4
    cols = xp.shape[1] // n_chunk
    for c in range(n_chunk):
        xc = xp[:, c * cols:(c + 1) * cols]
        hs = jnp.maximum(
            jnp.dot(w1t_ref[...], xc, preferred_element_type=jnp.float32)
            + b1c_ref[...], 0.0)
        z = jnp.dot(w2t_ref[...], hs, preferred_element_type=jnp.float32) + bs2_ref[0, 0]
        chunks.append(1.0 / (1.0 + jnp.exp(-z)))
    score = jnp.concatenate(chunks, axis=1).reshape(n_s, n_t)
    score_ref[0] = score

    iota_t = lax.broadcasted_iota(jnp.int32, (n_s, n_t), 1).astype(jnp.float32)
    col_k = lax.broadcasted_iota(jnp.int32, (n_s, 16), 1).astype(jnp.float32)
    bos = w_col[1:n_s + 1, :] > pre[1:n_s + 1, :]

    smax = score
    smin = score
    tp = jnp.zeros((n_s, 16), dtype=jnp.float32)
    for k in range(16):
        mx = jnp.max(smax, axis=1, keepdims=True)
        sell_idx = jnp.min(jnp.where(smax == mx, iota_t, jnp.float32(n_t)),
                           axis=1, keepdims=True)
        smax = jnp.where(iota_t == sell_idx, jnp.float32(-jnp.inf), smax)
        mn = jnp.min(smin, axis=1, keepdims=True)
        buy_idx = jnp.min(jnp.where(smin == mn, iota_t, jnp.float32(n_t)),
                          axis=1, keepdims=True)
        smin = jnp.where(iota_t == buy_idx, jnp.float32(jnp.inf), smin)
        choice = jnp.where(bos, buy_idx, sell_idx)
        tp = jnp.where(col_k == jnp.float32(k), choice, tp)
    tp_ref[0] = tp.astype(jnp.int32)


def kernel(micro_price, pre_w, params):
    b, s, t, f = micro_price.shape
    n = s + 1
    n_pad = 512

    mpt = micro_price.transpose(0, 3, 1, 2).reshape(b, f, s * t)
    prew_pad = jnp.pad(pre_w, ((0, 0), (0, n_pad - n)))[..., None]

    wall = jnp.stack([p["W"] for p in params["gat_heads"]])
    wstack = wall.reshape(4, t, f, 64).transpose(0, 2, 1, 3).reshape(4, t * f, 64)
    aall = jnp.stack([p["a"] for p in params["gat_heads"]])
    a1 = aall[:, :64]
    a2 = aall[:, 64:]
    wout4 = params["gat_out"]["W"].reshape(4, 64, 64)
    ao1 = params["gat_out"]["a"][:64]
    ao2 = params["gat_out"]["a"][64:]
    wm = params["w_mlp"]
    wm1 = wm[0]["W"][:64]
    wm1p = wm[0]["W"][64:65]
    bm1 = wm[0]["b"][None, :]
    wm2 = wm[1]["W"]
    bm2 = wm[1]["b"].reshape(1, 1)
    sc = params["score_mlp"]
    w1t = sc[0]["W"].T
    b1c = sc[0]["b"][:, None]
    w2t = sc[1]["W"].T
    bs2 = sc[1]["b"].reshape(1, 1)

    def _full(shape):
        return pl.BlockSpec(shape, lambda *_: (0,) * len(shape))

    w3, score, trading_points = pl.pallas_call(
        _body,
        grid=(b,),
        in_specs=[
            pl.BlockSpec((1, f, s * t), lambda i: (i, 0, 0)),
            pl.BlockSpec((1, n_pad, 1), lambda i: (i, 0, 0)),
            _full((4, t * f, 64)), _full((4, 64, 1)), _full((4, 64, 1)),
            _full((4, 64, 64)), _full((64, 1)), _full((64, 1)),
            _full((64, 64)), _full((1, 64)), _full((1, 64)),
            _full((64, 1)), _full((1, 1)),
            _full((64, f)), _full((64, 1)), _full((1, 64)), _full((1, 1)),
        ],
        out_specs=[
            pl.BlockSpec((1, n_pad, 1), lambda i: (i, 0, 0)),
            pl.BlockSpec((1, s, t), lambda i: (i, 0, 0)),
            pl.BlockSpec((1, s, 16), lambda i: (i, 0, 0)),
        ],
        out_shape=[
            jax.ShapeDtypeStruct((b, n_pad, 1), jnp.float32),
            jax.ShapeDtypeStruct((b, s, t), jnp.float32),
            jax.ShapeDtypeStruct((b, s, 16), jnp.int32),
        ],
        interpret=_INTERPRET,
    )(mpt, prew_pad, wstack, a1, a2, wout4, ao1, ao2,
      wm1, wm1p, bm1, wm2, bm2, w1t, b1c, w2t, bs2)

    w = w3[:, :n, 0]
    return w, trading_points, score
